# baseline (speedup 1.0000x reference)
import functools

import jax
import jax.numpy as jnp
from jax import lax
from jax.experimental import pallas as pl
from jax.experimental.pallas import tpu as pltpu
from jax.experimental.pallas import tpu_sc as plsc

_CHUNK = 128
_NBUF = 8
_TBLK = 2048
_LANES = 16


def _tc_transpose(wt, v):
  c = wt.shape[0]
  grid = (v + _TBLK - 1) // _TBLK

  def body(in_ref, out_ref):
    y = in_ref[...].T.reshape(_TBLK // 4, 4, c)
    for q in range(4):
      out_ref[:, q * c:(q + 1) * c] = y[:, q, :]

  return pl.pallas_call(
      body,
      grid=(grid,),
      in_specs=[pl.BlockSpec((c, _TBLK), lambda g: (0, g))],
      out_specs=pl.BlockSpec((_TBLK // 4, 4 * c), lambda g: (g, 0)),
      out_shape=jax.ShapeDtypeStruct((v // 4, 4 * c), jnp.float32),
  )(wt)


@functools.cache
def _build(n_chunks, d, nc, f_dim, bt_dim):
  assert d == 32
  assert n_chunks % _NBUF == 0 and n_chunks // _NBUF >= 2
  n_outer = n_chunks // _NBUF
  mesh = plsc.VectorSubcoreMesh(core_axis_name="c", subcore_axis_name="s")

  @functools.partial(
      pl.kernel,
      out_type=jax.ShapeDtypeStruct((f_dim, 4, bt_dim, 8 * 128),
                                    jnp.float32),
      mesh=mesh,
      scratch_types=[
          pltpu.VMEM((n_chunks, _CHUNK), jnp.int32),
          pltpu.VMEM((_NBUF, _CHUNK, 32), jnp.float32),
          pltpu.VMEM((2, 4, 8 * 128), jnp.float32),
          pltpu.SemaphoreType.DMA((_NBUF,)),
          pltpu.SemaphoreType.DMA((2,)),
      ],
      compiler_params=pltpu.CompilerParams(use_tc_tiling_on_sc=False,
                                           needs_layout_passes=False),
  )
  def gather_kernel(table_hbm, idx_hbm, out_hbm, idx_v, rv, tv, gsem, tsem):
    wid = lax.axis_index("s") * nc + lax.axis_index("c")
    pltpu.sync_copy(idx_hbm.at[wid], idx_v)

    for b in range(_NBUF):
      pltpu.async_copy(table_hbm.at[idx_v.at[b]], rv.at[b], gsem.at[b])

    lane = lax.broadcasted_iota(jnp.int32, (_LANES,), 0)
    rows_k = [lane + k * _LANES for k in range(8)]

    cols_c = [lane * 0 + cc for cc in range(32)]

    def transpose_chunk(b, p):
      def k_body(k, carry):
        rows = lane + k * _LANES
        base = k * _LANES
        for cc in range(32):
          vec = plsc.load_gather(rv.at[b], [rows, cols_c[cc]])
          tv[p, cc // 8, pl.ds((cc % 8) * 128 + base, _LANES)] = vec
        return carry

      lax.fori_loop(0, 8, k_body, 0, unroll=False)

    def store_chunk(j, p):
      u = wid * n_chunks + j
      f = u // bt_dim
      bt = lax.rem(u, bt_dim)
      pltpu.async_copy(tv.at[p], out_hbm.at[f, :, bt], tsem.at[p])

    def wait_tv(p):
      pltpu.make_async_copy(tv.at[p], out_hbm.at[0, :, 0], tsem.at[p]).wait()

    for b in range(_NBUF):
      pltpu.make_async_copy(table_hbm.at[idx_v.at[b]], rv.at[b],
                            gsem.at[b]).wait()
      if b >= 2:
        wait_tv(b % 2)
      transpose_chunk(b, b % 2)
      store_chunk(b, b % 2)
      pltpu.async_copy(table_hbm.at[idx_v.at[b + _NBUF]], rv.at[b],
                       gsem.at[b])

    def body(o, carry):
      for b in range(_NBUF):
        j = o * _NBUF + b
        pltpu.make_async_copy(table_hbm.at[idx_v.at[j]], rv.at[b],
                              gsem.at[b]).wait()
        wait_tv(b % 2)
        transpose_chunk(b, b % 2)
        store_chunk(j, b % 2)
        pltpu.async_copy(table_hbm.at[idx_v.at[j + _NBUF]], rv.at[b],
                         gsem.at[b])
      return carry

    lax.fori_loop(1, n_outer - 1, body, 0)

    for b in range(_NBUF):
      j = (n_outer - 1) * _NBUF + b
      pltpu.make_async_copy(table_hbm.at[idx_v.at[j]], rv.at[b],
                            gsem.at[b]).wait()
      wait_tv(b % 2)
      transpose_chunk(b, b % 2)
      store_chunk(j, b % 2)

    wait_tv(0)
    wait_tv(1)

  return gather_kernel


def kernel(x, weights):
  b, f = x.shape
  v, d = weights.shape
  mesh = plsc.VectorSubcoreMesh(core_axis_name="c", subcore_axis_name="s")
  nw = mesh.num_cores * mesh.num_subcores
  bt_dim = b // _CHUNK
  n_chunks = (f * bt_dim) // nw
  table2 = _tc_transpose(weights.T, v).reshape(4 * (v // 4), d)
  idx = x.T.reshape(f * bt_dim, _CHUNK).reshape(nw, n_chunks, _CHUNK)
  idx = idx.astype(jnp.int32)
  out4 = _build(n_chunks, d, mesh.num_cores, f, bt_dim)(table2, idx)
  out5 = out4.reshape(f, 4, bt_dim, 8, 128)
  return out5.transpose(2, 4, 0, 1, 3).reshape(b, f, d)


# speedup vs baseline: 1.0548x; 1.0548x over previous
import functools

import jax
import jax.numpy as jnp
from jax import lax
from jax.experimental import pallas as pl
from jax.experimental.pallas import tpu as pltpu
from jax.experimental.pallas import tpu_sc as plsc

_CHUNK = 128
_NBUF = 8
_TBLK = 2048


def _tc_transpose(wt, v):
  c = wt.shape[0]
  grid = (v + _TBLK - 1) // _TBLK

  def body(in_ref, out_ref):
    y = in_ref[...].T.reshape(_TBLK // 4, 4, c)
    for q in range(4):
      out_ref[:, q * c:(q + 1) * c] = y[:, q, :]

  return pl.pallas_call(
      body,
      grid=(grid,),
      in_specs=[pl.BlockSpec((c, _TBLK), lambda g: (0, g))],
      out_specs=pl.BlockSpec((_TBLK // 4, 4 * c), lambda g: (g, 0)),
      out_shape=jax.ShapeDtypeStruct((v // 4, 4 * c), jnp.float32),
  )(wt)


@functools.cache
def _build(n_workers, n_chunks, d, nc):
  assert n_chunks % _NBUF == 0 and n_chunks // _NBUF >= 2
  n_outer = n_chunks // _NBUF
  mesh = plsc.VectorSubcoreMesh(core_axis_name="c", subcore_axis_name="s")

  @functools.partial(
      pl.kernel,
      out_type=jax.ShapeDtypeStruct((n_workers, n_chunks, _CHUNK, d),
                                    jnp.float32),
      mesh=mesh,
      scratch_types=[
          pltpu.VMEM((n_chunks, _CHUNK), jnp.int32),
          pltpu.VMEM((_NBUF, _CHUNK, d), jnp.float32),
          pltpu.SemaphoreType.DMA((_NBUF,)),
      ],
      compiler_params=pltpu.CompilerParams(use_tc_tiling_on_sc=False),
  )
  def gather_kernel(table_hbm, idx_hbm, out_hbm, idx_v, rv, gsem):
    wid = lax.axis_index("s") * nc + lax.axis_index("c")
    pltpu.sync_copy(idx_hbm.at[wid], idx_v)

    def src(j):
      return table_hbm.at[idx_v.at[j]]

    for b in range(_NBUF):
      pltpu.async_copy(src(b), rv.at[b], gsem.at[b])

    def body(o, carry):
      for b in range(_NBUF):
        j = o * _NBUF + b
        pltpu.make_async_copy(src(j), rv.at[b], gsem.at[b]).wait()
        pltpu.sync_copy(rv.at[b], out_hbm.at[wid, j])
        pltpu.async_copy(src(j + _NBUF), rv.at[b], gsem.at[b])
      return carry

    lax.fori_loop(0, n_outer - 1, body, 0)

    for b in range(_NBUF):
      j = (n_outer - 1) * _NBUF + b
      pltpu.make_async_copy(src(j), rv.at[b], gsem.at[b]).wait()
      pltpu.sync_copy(rv.at[b], out_hbm.at[wid, j])

  return gather_kernel


def kernel(x, weights):
  b, f = x.shape
  v, d = weights.shape
  n = b * f
  mesh = plsc.VectorSubcoreMesh(core_axis_name="c", subcore_axis_name="s")
  nw = mesh.num_cores * mesh.num_subcores
  table2 = _tc_transpose(weights.T, v).reshape(4 * (v // 4), d)
  idx = x.reshape(n).astype(jnp.int32)
  pad = (-n) % (nw * _CHUNK)
  if pad:
    idx = jnp.concatenate([idx, jnp.zeros((pad,), jnp.int32)])
  n_chunks = (n + pad) // (nw * _CHUNK)
  idx = idx.reshape(nw, n_chunks, _CHUNK)
  out = _build(nw, n_chunks, d, mesh.num_cores)(table2, idx)
  return out.reshape((n + pad), d)[:n].reshape(b, f, d)
